# baseline (speedup 1.0000x reference)
import functools

import jax
import jax.numpy as jnp
import numpy as np
from jax import lax
from jax.experimental import pallas as pl
from jax.experimental.pallas import tpu as pltpu
from jax.experimental.pallas import tpu_sc as plsc

SCALE = 0.25
PH = PW = 7
S = 2
C = 64
RP = 1024
LANES = 784
BINS = PH * PW
NR = RP * BINS
NW = 32
CHUNK_ROWS = 8
CHUNKS = NR // (NW * CHUNK_ROWS)


def _lane_consts():
    l = np.arange(LANES)
    p = l // 16
    q = l % 16
    py, px = p // PW, p % PW
    ygrp, xgrp = q // 4, q % 4
    sy, ky_i = ygrp // 2, ygrp % 2
    sx, kx_i = xgrp // 2, xgrp % 2
    offy = (sy + 0.5) / S
    offx = (sx + 0.5) / S
    ay = (py + offy).astype(np.float32)
    ax = (px + offx).astype(np.float32)
    ky = ky_i.astype(np.float32)
    kx = kx_i.astype(np.float32)
    return tuple(a.reshape(1, LANES) for a in (ay, ax, ky, kx))


_AY, _AX, _KY, _KX = _lane_consts()


def _coords_body(rois_ref, ay_ref, ax_ref, ky_ref, kx_ref,
                 idx1_ref, w1_ref, idx2_ref, w2_ref):
    r = rois_ref[...]
    b = r[:, 0:1].astype(jnp.int32)
    x1 = r[:, 1:2] * SCALE
    y1 = r[:, 2:3] * SCALE
    x2 = r[:, 3:4] * SCALE
    y2 = r[:, 4:5] * SCALE
    bw = jnp.maximum(x2 - x1, 1.0) * (1.0 / PW)
    bh = jnp.maximum(y2 - y1, 1.0) * (1.0 / PH)
    ay, ax = ay_ref[...], ax_ref[...]
    ky, kx = ky_ref[...], kx_ref[...]
    kyi = ky.astype(jnp.int32)
    kxi = kx.astype(jnp.int32)
    y = y1 + ay * bh
    x = x1 + ax * bw

    def stage(H, W, idx_ref, w_ref):
        yc = jnp.clip(y, 0.0, float(H - 1))
        xc = jnp.clip(x, 0.0, float(W - 1))
        y0f = jnp.floor(yc)
        x0f = jnp.floor(xc)
        ly = yc - y0f
        lx = xc - x0f
        wy = ky * ly + (1.0 - ky) * (1.0 - ly)
        wx = kx * lx + (1.0 - kx) * (1.0 - lx)
        y0 = y0f.astype(jnp.int32)
        x0 = x0f.astype(jnp.int32)
        yi = jnp.minimum(y0 + kyi, H - 1)
        xi = jnp.minimum(x0 + kxi, W - 1)
        idx_ref[...] = b * (H * W) + yi * W + xi
        w_ref[...] = wy * wx * (1.0 / (S * S))

    stage(200, 200, idx1_ref, w1_ref)
    stage(PH, PW, idx2_ref, w2_ref)


def _coords(rois_p):
    grid = RP // 8
    cspec = pl.BlockSpec((1, LANES), lambda i: (0, 0))
    ospec = pl.BlockSpec((8, LANES), lambda i: (i, 0))
    return pl.pallas_call(
        _coords_body,
        grid=(grid,),
        in_specs=[pl.BlockSpec((8, 8), lambda i: (i, 0))] + [cspec] * 4,
        out_specs=[ospec] * 4,
        out_shape=[
            jax.ShapeDtypeStruct((RP, LANES), jnp.int32),
            jax.ShapeDtypeStruct((RP, LANES), jnp.float32),
            jax.ShapeDtypeStruct((RP, LANES), jnp.int32),
            jax.ShapeDtypeStruct((RP, LANES), jnp.float32),
        ],
    )(rois_p, jnp.asarray(_AY), jnp.asarray(_AX), jnp.asarray(_KY),
      jnp.asarray(_KX))


NSUB = 1
SUBROWS = CHUNK_ROWS * 16 // NSUB
NPASS = 4
PC = C // NPASS
TROWS = 2 * 200 * 200


HC = C // 2
SROWS = NR // 16
SCHUNKS = SROWS // CHUNK_ROWS
NSEC = 7
SECC = SCHUNKS // NSEC


def _gather_sum_bf_kernel(table_hbm, idx_hbm, w_hbm, out_hbm,
                          spm, idx_v, w_v, g_a, g_b, ovm, sem_a, sem_b):
    sid = lax.axis_index("s")
    cid = lax.axis_index("c")
    stage_rows = TROWS // 16

    for core in range(2):
        @pl.when(cid == core)
        def _(core=core):
            pltpu.sync_copy(
                table_hbm.at[pl.ds(sid * stage_rows, stage_rows),
                             pl.ds(core * (HC // 2), HC // 2)],
                spm.at[pl.ds(sid * stage_rows, stage_rows)])

    plsc.subcore_barrier()

    def fire(j, g, sem):
        pltpu.async_copy(spm.at[idx_v.at[j]], g, sem)

    def drain(j, g, sem):
        pltpu.make_async_copy(spm.at[idx_v.at[j]], g, sem).wait()

    def compute(j, g):
        for r in range(CHUNK_ROWS):
            pe = [jnp.zeros((16,), jnp.float32) for _ in range(2)]
            po = [jnp.zeros((16,), jnp.float32) for _ in range(2)]
            wrow = w_v[j, pl.ds(r * 16, 16)]
            for q in range(16):
                wv = jnp.full((16,), wrow[q], jnp.float32)
                vi = g[r * 16 + q, :]
                fe = plsc.bitcast(jnp.left_shift(vi, 16), jnp.float32)
                fo = plsc.bitcast(
                    jnp.bitwise_and(vi, jnp.int32(-65536)), jnp.float32)
                pe[q % 2] = pe[q % 2] + wv * fe
                po[q % 2] = po[q % 2] + wv * fo
            obase = (j * CHUNK_ROWS + r) * HC
            ovm[pl.ds(obase, 16)] = pe[0] + pe[1]
            ovm[pl.ds(obase + 16, 16)] = po[0] + po[1]

    for sec in range(NSEC):
        pltpu.sync_copy(idx_hbm.at[sid, pl.ds(sec * SECC, SECC)], idx_v)
        pltpu.sync_copy(w_hbm.at[sid, pl.ds(sec * SECC, SECC)], w_v)

        fire(0, g_a, sem_a)

        def body(jj, carry):
            j = jj * 2
            fire(j + 1, g_b, sem_b)
            drain(j, g_a, sem_a)
            compute(j, g_a)

            @pl.when(jj < SECC // 2 - 1)
            def _():
                fire(j + 2, g_a, sem_a)

            drain(j + 1, g_b, sem_b)
            compute(j + 1, g_b)
            return carry

        lax.fori_loop(0, SECC // 2, body, 0)
        pltpu.sync_copy(ovm, out_hbm.at[cid, pl.ds(
            (sid * SROWS + sec * SECC * CHUNK_ROWS) * HC,
            SECC * CHUNK_ROWS * HC)])


def _gather_sum_bf(table_bf, idx, w):
    mesh = plsc.VectorSubcoreMesh(core_axis_name="c", subcore_axis_name="s")
    oshape = jax.ShapeDtypeStruct((2, NR * HC), jnp.float32)
    kfn = functools.partial(
        pl.kernel,
        mesh=mesh,
        compiler_params=pltpu.CompilerParams(use_tc_tiling_on_sc=False,
                                             needs_layout_passes=False),
        out_type=oshape,
        scratch_types=[
            pltpu.VMEM_SHARED((TROWS, HC // 2), jnp.int32),
            pltpu.VMEM((SECC, 128), jnp.int32),
            pltpu.VMEM((SECC, 128), jnp.float32),
            pltpu.VMEM((128, HC // 2), jnp.int32),
            pltpu.VMEM((128, HC // 2), jnp.int32),
            pltpu.VMEM((SECC * CHUNK_ROWS * HC,), jnp.float32),
            pltpu.SemaphoreType.DMA,
            pltpu.SemaphoreType.DMA,
        ],
    )(_gather_sum_bf_kernel)
    return kfn(table_bf, idx, w)


TBL2 = 2 * BINS * HC


def _stage2_kernel(t0, t1, idx_hbm, w_hbm, out_hbm, tb, idx_v, w_v, ovm):
    nc = 2
    hchunks = CHUNKS // 2
    wid = lax.axis_index("s") * nc + lax.axis_index("c")
    for p, tp in enumerate((t0, t1)):
        pltpu.sync_copy(tp.at[pl.ds(0, TBL2)], tb.at[pl.ds(p * TBL2, TBL2)])
    row0 = wid * (CHUNKS * CHUNK_ROWS)

    for h in range(2):
        pltpu.sync_copy(idx_hbm.at[wid, pl.ds(h * hchunks * NSUB,
                                              hchunks * NSUB)], idx_v)
        pltpu.sync_copy(w_hbm.at[wid, pl.ds(h * hchunks, hchunks)], w_v)

        def body(j, carry):
            for r in range(CHUNK_ROWS):
                irow = idx_v[j, pl.ds(r * 16, 16)]
                wrow = w_v[j, pl.ds(r * 16, 16)]
                part = [jnp.zeros((16,), jnp.float32) for _ in range(8)]
                for q in range(16):
                    base = irow[q] * HC
                    wv = jnp.full((16,), wrow[q], jnp.float32)
                    for k in range(4):
                        off = (k // 2) * TBL2 + (k % 2) * 16
                        part[k * 2 + q % 2] = (part[k * 2 + q % 2]
                                               + wv * tb[pl.ds(base + off,
                                                               16)])
                for k in range(4):
                    ovm[pl.ds((j * CHUNK_ROWS + r) * C + k * 16, 16)] = (
                        part[k * 2] + part[k * 2 + 1])
            return carry

        lax.fori_loop(0, hchunks, body, 0)
        pltpu.sync_copy(ovm, out_hbm.at[pl.ds(
            (row0 + h * hchunks * CHUNK_ROWS) * C,
            hchunks * CHUNK_ROWS * C)])


def _stage2(parts, idx, w):
    mesh = plsc.VectorSubcoreMesh(core_axis_name="c", subcore_axis_name="s")
    hrows = (CHUNKS // 2) * CHUNK_ROWS
    kfn = functools.partial(
        pl.kernel,
        mesh=mesh,
        compiler_params=pltpu.CompilerParams(use_tc_tiling_on_sc=False),
        out_type=jax.ShapeDtypeStruct((NR * C,), jnp.float32),
        scratch_types=[
            pltpu.VMEM((2 * TBL2,), jnp.float32),
            pltpu.VMEM((CHUNKS // 2 * NSUB, SUBROWS), jnp.int32),
            pltpu.VMEM((CHUNKS // 2, 128), jnp.float32),
            pltpu.VMEM((hrows * C,), jnp.float32),
        ],
    )(_stage2_kernel)
    return kfn(*parts, idx, w)


def _pack(a, cols=128):
    return a.reshape(NW, (CHUNKS * 128) // cols, cols)


def _pack16(a):
    return a.reshape(16, SCHUNKS, 128)


def _chan_perm():
    ch = np.arange(C)
    p, r = ch // HC, ch % HC
    return ((p * 2 + r % 2) * 16 + r // 2).astype(np.int32)


_PERM = _chan_perm()


def kernel(input, rois):
    feat = jnp.transpose(input, (0, 2, 3, 1)).reshape(-1, C)
    feat_bf = feat.astype(jnp.bfloat16)
    feat_i = lax.bitcast_convert_type(
        feat_bf.reshape(TROWS, C // 2, 2), jnp.int32)
    rois_p = jnp.zeros((RP, 8), jnp.float32).at[:rois.shape[0], :5].set(rois)
    idx1, w1, idx2, w2 = _coords(rois_p)
    o1_pair = _gather_sum_bf(feat_i, _pack16(idx1), _pack16(w1))
    o2 = _stage2((o1_pair[0], o1_pair[1]), _pack(idx2, SUBROWS), _pack(w2)).reshape(NR, C)
    o2 = o2[:, jnp.asarray(_PERM)]
    out = o2[: rois.shape[0] * BINS].reshape(-1, BINS, C)
    return jnp.transpose(out, (0, 2, 1)).reshape(-1, C, PH, PW)


# speedup vs baseline: 1.0016x; 1.0016x over previous
import functools

import jax
import jax.numpy as jnp
import numpy as np
from jax import lax
from jax.experimental import pallas as pl
from jax.experimental.pallas import tpu as pltpu
from jax.experimental.pallas import tpu_sc as plsc

SCALE = 0.25
PH = PW = 7
S = 2
C = 64
RP = 1024
LANES = 784
BINS = PH * PW
NR = RP * BINS
NW = 32
CHUNK_ROWS = 8
CHUNKS = NR // (NW * CHUNK_ROWS)


def _lane_consts():
    l = np.arange(LANES)
    p = l // 16
    q = l % 16
    py, px = p // PW, p % PW
    ygrp, xgrp = q // 4, q % 4
    sy, ky_i = ygrp // 2, ygrp % 2
    sx, kx_i = xgrp // 2, xgrp % 2
    offy = (sy + 0.5) / S
    offx = (sx + 0.5) / S
    ay = (py + offy).astype(np.float32)
    ax = (px + offx).astype(np.float32)
    ky = ky_i.astype(np.float32)
    kx = kx_i.astype(np.float32)
    return tuple(a.reshape(1, LANES) for a in (ay, ax, ky, kx))


_AY, _AX, _KY, _KX = _lane_consts()


def _coords_body(rois_ref, ay_ref, ax_ref, ky_ref, kx_ref,
                 idx1_ref, w1_ref, idx2_ref, w2_ref):
    r = rois_ref[...]
    b = r[:, 0:1].astype(jnp.int32)
    x1 = r[:, 1:2] * SCALE
    y1 = r[:, 2:3] * SCALE
    x2 = r[:, 3:4] * SCALE
    y2 = r[:, 4:5] * SCALE
    bw = jnp.maximum(x2 - x1, 1.0) * (1.0 / PW)
    bh = jnp.maximum(y2 - y1, 1.0) * (1.0 / PH)
    ay, ax = ay_ref[...], ax_ref[...]
    ky, kx = ky_ref[...], kx_ref[...]
    kyi = ky.astype(jnp.int32)
    kxi = kx.astype(jnp.int32)
    y = y1 + ay * bh
    x = x1 + ax * bw

    def stage(H, W, idx_ref, w_ref):
        yc = jnp.clip(y, 0.0, float(H - 1))
        xc = jnp.clip(x, 0.0, float(W - 1))
        y0f = jnp.floor(yc)
        x0f = jnp.floor(xc)
        ly = yc - y0f
        lx = xc - x0f
        wy = ky * ly + (1.0 - ky) * (1.0 - ly)
        wx = kx * lx + (1.0 - kx) * (1.0 - lx)
        y0 = y0f.astype(jnp.int32)
        x0 = x0f.astype(jnp.int32)
        yi = jnp.minimum(y0 + kyi, H - 1)
        xi = jnp.minimum(x0 + kxi, W - 1)
        idx_ref[...] = b * (H * W) + yi * W + xi
        w_ref[...] = wy * wx * (1.0 / (S * S))

    stage(200, 200, idx1_ref, w1_ref)
    stage(PH, PW, idx2_ref, w2_ref)


def _coords(rois_p):
    grid = RP // 8
    cspec = pl.BlockSpec((1, LANES), lambda i: (0, 0))
    ospec = pl.BlockSpec((8, LANES), lambda i: (i, 0))
    return pl.pallas_call(
        _coords_body,
        grid=(grid,),
        in_specs=[pl.BlockSpec((8, 8), lambda i: (i, 0))] + [cspec] * 4,
        out_specs=[ospec] * 4,
        out_shape=[
            jax.ShapeDtypeStruct((RP, LANES), jnp.int32),
            jax.ShapeDtypeStruct((RP, LANES), jnp.float32),
            jax.ShapeDtypeStruct((RP, LANES), jnp.int32),
            jax.ShapeDtypeStruct((RP, LANES), jnp.float32),
        ],
    )(rois_p, jnp.asarray(_AY), jnp.asarray(_AX), jnp.asarray(_KY),
      jnp.asarray(_KX))


NSUB = 1
SUBROWS = CHUNK_ROWS * 16 // NSUB
NPASS = 4
PC = C // NPASS
TROWS = 2 * 200 * 200


HC = C // 2
SROWS = NR // 16
SCHUNKS = SROWS // CHUNK_ROWS
NSEC = 7
SECC = SCHUNKS // NSEC


def _gather_sum_bf_kernel(table_hbm, idx_hbm, w_hbm, out_hbm,
                          spm, idx_v, w_v, g_a, g_b, ovm, sem_a, sem_b):
    sid = lax.axis_index("s")
    cid = lax.axis_index("c")
    stage_rows = TROWS // 16

    for core in range(2):
        @pl.when(cid == core)
        def _(core=core):
            pltpu.sync_copy(
                table_hbm.at[pl.ds(sid * stage_rows, stage_rows),
                             pl.ds(core * (HC // 2), HC // 2)],
                spm.at[pl.ds(sid * stage_rows, stage_rows)])

    plsc.subcore_barrier()

    def fire(j, g, sem):
        pltpu.async_copy(spm.at[idx_v.at[j]], g, sem)

    def drain(j, g, sem):
        pltpu.make_async_copy(spm.at[idx_v.at[j]], g, sem).wait()

    def compute(j, g):
        for r in range(CHUNK_ROWS):
            pe = [jnp.zeros((16,), jnp.float32) for _ in range(2)]
            po = [jnp.zeros((16,), jnp.float32) for _ in range(2)]
            wrow = w_v[j, pl.ds(r * 16, 16)]
            for q in range(16):
                wv = jnp.full((16,), wrow[q], jnp.float32)
                vi = g[r * 16 + q, :]
                fe = plsc.bitcast(jnp.left_shift(vi, 16), jnp.float32)
                fo = plsc.bitcast(
                    jnp.bitwise_and(vi, jnp.int32(-65536)), jnp.float32)
                pe[q % 2] = pe[q % 2] + wv * fe
                po[q % 2] = po[q % 2] + wv * fo
            obase = (j * CHUNK_ROWS + r) * HC
            ovm[pl.ds(obase, 16)] = pe[0] + pe[1]
            ovm[pl.ds(obase + 16, 16)] = po[0] + po[1]

    for sec in range(NSEC):
        pltpu.sync_copy(idx_hbm.at[sid, pl.ds(sec * SECC, SECC)], idx_v)
        pltpu.sync_copy(w_hbm.at[sid, pl.ds(sec * SECC, SECC)], w_v)

        fire(0, g_a, sem_a)

        def body(jj, carry):
            j = jj * 2
            fire(j + 1, g_b, sem_b)
            drain(j, g_a, sem_a)
            compute(j, g_a)

            @pl.when(jj < SECC // 2 - 1)
            def _():
                fire(j + 2, g_a, sem_a)

            drain(j + 1, g_b, sem_b)
            compute(j + 1, g_b)
            return carry

        lax.fori_loop(0, SECC // 2, body, 0)
        pltpu.sync_copy(ovm, out_hbm.at[cid, pl.ds(
            (sid * SROWS + sec * SECC * CHUNK_ROWS) * HC,
            SECC * CHUNK_ROWS * HC)])


def _gather_sum_bf(table_bf, idx, w):
    mesh = plsc.VectorSubcoreMesh(core_axis_name="c", subcore_axis_name="s")
    oshape = jax.ShapeDtypeStruct((2, NR * HC), jnp.float32)
    kfn = functools.partial(
        pl.kernel,
        mesh=mesh,
        compiler_params=pltpu.CompilerParams(use_tc_tiling_on_sc=False,
                                             needs_layout_passes=False),
        out_type=oshape,
        scratch_types=[
            pltpu.VMEM_SHARED((TROWS, HC // 2), jnp.int32),
            pltpu.VMEM((SECC, 128), jnp.int32),
            pltpu.VMEM((SECC, 128), jnp.float32),
            pltpu.VMEM((128, HC // 2), jnp.int32),
            pltpu.VMEM((128, HC // 2), jnp.int32),
            pltpu.VMEM((SECC * CHUNK_ROWS * HC,), jnp.float32),
            pltpu.SemaphoreType.DMA,
            pltpu.SemaphoreType.DMA,
        ],
    )(_gather_sum_bf_kernel)
    return kfn(table_bf, idx, w)


TBL2 = 2 * BINS * HC


def _stage2_kernel(t0, t1, idx_hbm, w_hbm, out_hbm, tb, idx_v, w_v, ovm):
    nc = 2
    hchunks = CHUNKS // 2
    wid = lax.axis_index("s") * nc + lax.axis_index("c")
    for p, tp in enumerate((t0, t1)):
        pltpu.sync_copy(tp.at[pl.ds(0, TBL2)], tb.at[pl.ds(p * TBL2, TBL2)])
    row0 = wid * (CHUNKS * CHUNK_ROWS)

    for h in range(2):
        pltpu.sync_copy(idx_hbm.at[wid, pl.ds(h * hchunks * NSUB,
                                              hchunks * NSUB)], idx_v)
        pltpu.sync_copy(w_hbm.at[wid, pl.ds(h * hchunks, hchunks)], w_v)

        def body(j, carry):
            for r in range(CHUNK_ROWS):
                irow = idx_v[j, pl.ds(r * 16, 16)]
                wrow = w_v[j, pl.ds(r * 16, 16)]
                part = [jnp.zeros((16,), jnp.float32) for _ in range(8)]
                for q in range(16):
                    base = irow[q] * HC
                    wv = jnp.full((16,), wrow[q], jnp.float32)
                    for k in range(4):
                        off = (k // 2) * TBL2 + (k % 2) * 16
                        part[k * 2 + q % 2] = (part[k * 2 + q % 2]
                                               + wv * tb[pl.ds(base + off,
                                                               16)])
                for k in range(4):
                    ovm[pl.ds((j * CHUNK_ROWS + r) * C + k * 16, 16)] = (
                        part[k * 2] + part[k * 2 + 1])
            return carry

        lax.fori_loop(0, hchunks, body, 0)
        pltpu.sync_copy(ovm, out_hbm.at[pl.ds(
            (row0 + h * hchunks * CHUNK_ROWS) * C,
            hchunks * CHUNK_ROWS * C)])


def _stage2(parts, idx, w):
    mesh = plsc.VectorSubcoreMesh(core_axis_name="c", subcore_axis_name="s")
    hrows = (CHUNKS // 2) * CHUNK_ROWS
    kfn = functools.partial(
        pl.kernel,
        mesh=mesh,
        compiler_params=pltpu.CompilerParams(use_tc_tiling_on_sc=False),
        out_type=jax.ShapeDtypeStruct((NR * C,), jnp.float32),
        scratch_types=[
            pltpu.VMEM((2 * TBL2,), jnp.float32),
            pltpu.VMEM((CHUNKS // 2 * NSUB, SUBROWS), jnp.int32),
            pltpu.VMEM((CHUNKS // 2, 128), jnp.float32),
            pltpu.VMEM((hrows * C,), jnp.float32),
        ],
    )(_stage2_kernel)
    return kfn(*parts, idx, w)


def _pack(a, cols=128):
    return a.reshape(NW, (CHUNKS * 128) // cols, cols)


def _pack16(a):
    return a.reshape(16, SCHUNKS, 128)


def _chan_perm():
    ch = np.arange(C)
    p, r = ch // HC, ch % HC
    return ((p * 2 + r % 2) * 16 + r // 2).astype(np.int32)


_PERM = _chan_perm()


def kernel(input, rois):
    feat = jnp.transpose(input, (0, 2, 3, 1)).reshape(-1, C)
    lo_b = lax.bitcast_convert_type(
        feat[:, :HC].astype(jnp.bfloat16), jnp.uint16).astype(jnp.int32)
    hi_b = lax.bitcast_convert_type(
        feat[:, HC:].astype(jnp.bfloat16), jnp.uint16).astype(jnp.int32)
    feat_i = lo_b | (hi_b << 16)
    rois_p = jnp.zeros((RP, 8), jnp.float32).at[:rois.shape[0], :5].set(rois)
    idx1, w1, idx2, w2 = _coords(rois_p)
    o1_pair = _gather_sum_bf(feat_i, _pack16(idx1), _pack16(w1))
    o2 = _stage2((o1_pair[0], o1_pair[1]), _pack(idx2, SUBROWS),
                 _pack(w2)).reshape(NR, 4, 16)
    o2 = o2[:, jnp.asarray([0, 2, 1, 3]), :].reshape(NR, C)
    out = o2[: rois.shape[0] * BINS].reshape(-1, BINS, C)
    return jnp.transpose(out, (0, 2, 1)).reshape(-1, C, PH, PW)


# speedup vs baseline: 1.2489x; 1.2469x over previous
import functools

import jax
import jax.numpy as jnp
import numpy as np
from jax import lax
from jax.experimental import pallas as pl
from jax.experimental.pallas import tpu as pltpu
from jax.experimental.pallas import tpu_sc as plsc

SCALE = 0.25
PH = PW = 7
S = 2
C = 64
RP = 1024
LANES = 784
BINS = PH * PW
NR = RP * BINS
NW = 32
CHUNK_ROWS = 8
CHUNKS = NR // (NW * CHUNK_ROWS)


def _lane_consts():
    l = np.arange(LANES)
    p = l // 16
    q = l % 16
    py, px = p // PW, p % PW
    ygrp, xgrp = q // 4, q % 4
    sy, ky_i = ygrp // 2, ygrp % 2
    sx, kx_i = xgrp // 2, xgrp % 2
    offy = (sy + 0.5) / S
    offx = (sx + 0.5) / S
    ay = (py + offy).astype(np.float32)
    ax = (px + offx).astype(np.float32)
    ky = ky_i.astype(np.float32)
    kx = kx_i.astype(np.float32)
    return tuple(a.reshape(1, LANES) for a in (ay, ax, ky, kx))


_AY, _AX, _KY, _KX = _lane_consts()


def _coords_body(rois_ref, ay_ref, ax_ref, ky_ref, kx_ref,
                 idx1_ref, w1_ref, idx2_ref, w2_ref):
    r = rois_ref[...]
    b = r[:, 0:1].astype(jnp.int32)
    x1 = r[:, 1:2] * SCALE
    y1 = r[:, 2:3] * SCALE
    x2 = r[:, 3:4] * SCALE
    y2 = r[:, 4:5] * SCALE
    bw = jnp.maximum(x2 - x1, 1.0) * (1.0 / PW)
    bh = jnp.maximum(y2 - y1, 1.0) * (1.0 / PH)
    ay, ax = ay_ref[...], ax_ref[...]
    ky, kx = ky_ref[...], kx_ref[...]
    kyi = ky.astype(jnp.int32)
    kxi = kx.astype(jnp.int32)
    y = y1 + ay * bh
    x = x1 + ax * bw

    def stage(H, W, idx_ref, w_ref):
        yc = jnp.clip(y, 0.0, float(H - 1))
        xc = jnp.clip(x, 0.0, float(W - 1))
        y0f = jnp.floor(yc)
        x0f = jnp.floor(xc)
        ly = yc - y0f
        lx = xc - x0f
        wy = ky * ly + (1.0 - ky) * (1.0 - ly)
        wx = kx * lx + (1.0 - kx) * (1.0 - lx)
        y0 = y0f.astype(jnp.int32)
        x0 = x0f.astype(jnp.int32)
        yi = jnp.minimum(y0 + kyi, H - 1)
        xi = jnp.minimum(x0 + kxi, W - 1)
        idx_ref[...] = b * (H * W) + yi * W + xi
        w_ref[...] = wy * wx * (1.0 / (S * S))

    stage(200, 200, idx1_ref, w1_ref)
    stage(PH, PW, idx2_ref, w2_ref)


def _coords(rois_p):
    grid = RP // 8
    cspec = pl.BlockSpec((1, LANES), lambda i: (0, 0))
    ospec = pl.BlockSpec((8, LANES), lambda i: (i, 0))
    return pl.pallas_call(
        _coords_body,
        grid=(grid,),
        in_specs=[pl.BlockSpec((8, 8), lambda i: (i, 0))] + [cspec] * 4,
        out_specs=[ospec] * 4,
        out_shape=[
            jax.ShapeDtypeStruct((RP, LANES), jnp.int32),
            jax.ShapeDtypeStruct((RP, LANES), jnp.float32),
            jax.ShapeDtypeStruct((RP, LANES), jnp.int32),
            jax.ShapeDtypeStruct((RP, LANES), jnp.float32),
        ],
    )(rois_p, jnp.asarray(_AY), jnp.asarray(_AX), jnp.asarray(_KY),
      jnp.asarray(_KX))


NSUB = 1
SUBROWS = CHUNK_ROWS * 16 // NSUB
NPASS = 4
PC = C // NPASS
TROWS = 2 * 200 * 200


HC = C // 2
SROWS = NR // 16
SCHUNKS = SROWS // CHUNK_ROWS
NSEC = 7
SECC = SCHUNKS // NSEC


def _gather_sum_bf_kernel(table_hbm, idx_hbm, w_hbm, out_hbm,
                          spm, idx_v, w_v, g_a, g_b, ovm, sem_a, sem_b):
    sid = lax.axis_index("s")
    cid = lax.axis_index("c")
    stage_rows = TROWS // 16

    for core in range(2):
        @pl.when(cid == core)
        def _(core=core):
            pltpu.sync_copy(
                table_hbm.at[pl.ds(sid * stage_rows, stage_rows),
                             pl.ds(core * (HC // 2), HC // 2)],
                spm.at[pl.ds(sid * stage_rows, stage_rows)])

    plsc.subcore_barrier()

    def fire(j, g, sem):
        pltpu.async_copy(spm.at[idx_v.at[j]], g, sem)

    def drain(j, g, sem):
        pltpu.make_async_copy(spm.at[idx_v.at[j]], g, sem).wait()

    def compute(j, g):
        for r in range(CHUNK_ROWS):
            pe = [jnp.zeros((16,), jnp.float32) for _ in range(2)]
            po = [jnp.zeros((16,), jnp.float32) for _ in range(2)]
            wrow = w_v[j, pl.ds(r * 16, 16)]
            for q in range(16):
                wv = jnp.full((16,), wrow[q], jnp.float32)
                vi = g[r * 16 + q, :]
                fe = plsc.bitcast(jnp.left_shift(vi, 16), jnp.float32)
                fo = plsc.bitcast(
                    jnp.bitwise_and(vi, jnp.int32(-65536)), jnp.float32)
                pe[q % 2] = pe[q % 2] + wv * fe
                po[q % 2] = po[q % 2] + wv * fo
            obase = (j * CHUNK_ROWS + r) * HC
            ovm[pl.ds(obase, 16)] = pe[0] + pe[1]
            ovm[pl.ds(obase + 16, 16)] = po[0] + po[1]

    for sec in range(NSEC):
        pltpu.sync_copy(idx_hbm.at[sid, pl.ds(sec * SECC, SECC)], idx_v)
        pltpu.sync_copy(w_hbm.at[sid, pl.ds(sec * SECC, SECC)], w_v)

        fire(0, g_a, sem_a)

        def body(jj, carry):
            j = jj * 2
            fire(j + 1, g_b, sem_b)
            drain(j, g_a, sem_a)
            compute(j, g_a)

            @pl.when(jj < SECC // 2 - 1)
            def _():
                fire(j + 2, g_a, sem_a)

            drain(j + 1, g_b, sem_b)
            compute(j + 1, g_b)
            return carry

        lax.fori_loop(0, SECC // 2, body, 0)
        pltpu.sync_copy(ovm, out_hbm.at[cid, pl.ds(
            (sid * SROWS + sec * SECC * CHUNK_ROWS) * HC,
            SECC * CHUNK_ROWS * HC)])


def _gather_sum_bf(table_bf, idx, w):
    mesh = plsc.VectorSubcoreMesh(core_axis_name="c", subcore_axis_name="s")
    oshape = jax.ShapeDtypeStruct((2, NR * HC), jnp.float32)
    kfn = functools.partial(
        pl.kernel,
        mesh=mesh,
        compiler_params=pltpu.CompilerParams(use_tc_tiling_on_sc=False,
                                             needs_layout_passes=False),
        out_type=oshape,
        scratch_types=[
            pltpu.VMEM_SHARED((TROWS, HC // 2), jnp.int32),
            pltpu.VMEM((SECC, 128), jnp.int32),
            pltpu.VMEM((SECC, 128), jnp.float32),
            pltpu.VMEM((128, HC // 2), jnp.int32),
            pltpu.VMEM((128, HC // 2), jnp.int32),
            pltpu.VMEM((SECC * CHUNK_ROWS * HC,), jnp.float32),
            pltpu.SemaphoreType.DMA,
            pltpu.SemaphoreType.DMA,
        ],
    )(_gather_sum_bf_kernel)
    return kfn(table_bf, idx, w)


TBL2 = 2 * BINS * HC


def _stage2_kernel(t0, t1, idx_hbm, w_hbm, out_hbm, tb, idx_v, w_v, ovm):
    nc = 2
    hchunks = CHUNKS // 2
    wid = lax.axis_index("s") * nc + lax.axis_index("c")
    for p, tp in enumerate((t0, t1)):
        pltpu.sync_copy(tp, tb.at[pl.ds(p * TBL2, TBL2)])
    row0 = wid * (CHUNKS * CHUNK_ROWS)

    for h in range(2):
        pltpu.sync_copy(idx_hbm.at[wid, pl.ds(h * hchunks * NSUB,
                                              hchunks * NSUB)], idx_v)
        pltpu.sync_copy(w_hbm.at[wid, pl.ds(h * hchunks, hchunks)], w_v)

        def body(j, carry):
            for r in range(CHUNK_ROWS):
                irow = idx_v[j, pl.ds(r * 16, 16)]
                wrow = w_v[j, pl.ds(r * 16, 16)]
                part = [jnp.zeros((16,), jnp.float32) for _ in range(8)]
                for q in range(16):
                    base = irow[q] * HC
                    wv = jnp.full((16,), wrow[q], jnp.float32)
                    for k in range(4):
                        off = (k // 2) * TBL2 + (k % 2) * 16
                        part[k * 2 + q % 2] = (part[k * 2 + q % 2]
                                               + wv * tb[pl.ds(base + off,
                                                               16)])
                for k, blk in enumerate((0, 2, 1, 3)):
                    ovm[pl.ds((j * CHUNK_ROWS + r) * C + blk * 16, 16)] = (
                        part[k * 2] + part[k * 2 + 1])
            return carry

        lax.fori_loop(0, hchunks, body, 0)
        pltpu.sync_copy(ovm, out_hbm.at[pl.ds(
            (row0 + h * hchunks * CHUNK_ROWS) * C,
            hchunks * CHUNK_ROWS * C)])


def _stage2(parts, idx, w):
    mesh = plsc.VectorSubcoreMesh(core_axis_name="c", subcore_axis_name="s")
    hrows = (CHUNKS // 2) * CHUNK_ROWS
    kfn = functools.partial(
        pl.kernel,
        mesh=mesh,
        compiler_params=pltpu.CompilerParams(use_tc_tiling_on_sc=False),
        out_type=jax.ShapeDtypeStruct((NR * C,), jnp.float32),
        scratch_types=[
            pltpu.VMEM((2 * TBL2,), jnp.float32),
            pltpu.VMEM((CHUNKS // 2 * NSUB, SUBROWS), jnp.int32),
            pltpu.VMEM((CHUNKS // 2, 128), jnp.float32),
            pltpu.VMEM((hrows * C,), jnp.float32),
        ],
    )(_stage2_kernel)
    return kfn(*parts, idx, w)


def _pack(a, cols=128):
    return a.reshape(NW, (CHUNKS * 128) // cols, cols)


def _pack16(a):
    return a.reshape(16, SCHUNKS, 128)


def _chan_perm():
    ch = np.arange(C)
    p, r = ch // HC, ch % HC
    return ((p * 2 + r % 2) * 16 + r // 2).astype(np.int32)


_PERM = _chan_perm()


def kernel(input, rois):
    feat = jnp.transpose(input, (0, 2, 3, 1)).reshape(-1, C)
    lo_b = lax.bitcast_convert_type(
        feat[:, :HC].astype(jnp.bfloat16), jnp.uint16).astype(jnp.int32)
    hi_b = lax.bitcast_convert_type(
        feat[:, HC:].astype(jnp.bfloat16), jnp.uint16).astype(jnp.int32)
    feat_i = lo_b | (hi_b << 16)
    rois_p = jnp.zeros((RP, 8), jnp.float32).at[:rois.shape[0], :5].set(rois)
    idx1, w1, idx2, w2 = _coords(rois_p)
    o1_pair = _gather_sum_bf(feat_i, _pack16(idx1), _pack16(w1))
    o2 = _stage2((o1_pair[0, :TBL2], o1_pair[1, :TBL2]),
                 _pack(idx2, SUBROWS), _pack(w2)).reshape(NR, C)
    out = o2[: rois.shape[0] * BINS].reshape(-1, BINS, C)
    return jnp.transpose(out, (0, 2, 1)).reshape(-1, C, PH, PW)
